# baseline (speedup 1.0000x reference)
import functools

import jax
import jax.numpy as jnp
from jax import lax
from jax.experimental import pallas as pl
from jax.experimental.pallas import tpu as pltpu
from jax.experimental.pallas import tpu_sc as plsc

NUM_EXPERTS = 8
TOP_K = 2
HIDDEN = 1024
INTER = 1024
T = 2048
S = T * TOP_K
BM = 256
S_PAD = S + NUM_EXPERTS * BM
NT = S_PAD // BM

ROWS = 32
LANES = 128
NW = 32
SLOTS_PER_W = S // NW
CH = 64


def _lane_scan_incl(x):
    sh = 1
    col = lax.broadcasted_iota(jnp.int32, x.shape, 1)
    while sh < x.shape[1]:
        shifted = jnp.concatenate(
            [jnp.zeros((x.shape[0], sh), x.dtype), x[:, :-sh]], axis=1)
        x = x + jnp.where(col >= sh, shifted, 0)
        sh *= 2
    return x


def _sub_scan_incl(x):
    sh = 1
    row = lax.broadcasted_iota(jnp.int32, x.shape, 0)
    while sh < x.shape[0]:
        shifted = jnp.concatenate(
            [jnp.zeros((sh, x.shape[1]), x.dtype), x[:-sh, :]], axis=0)
        x = x + jnp.where(row >= sh, shifted, 0)
        sh *= 2
    return x


def _routing_body(ids_ref, wcol_ref, pos_ref, meta_ref, wbc_ref):
    ids = ids_ref[...]
    pos = jnp.zeros(ids.shape, jnp.int32)
    counts = []
    ranks = []
    for e in range(NUM_EXPERTS):
        m = (ids == e).astype(jnp.int32)
        incl = _lane_scan_incl(m)
        row_tot = jnp.broadcast_to(incl[:, LANES - 1:LANES], ids.shape)
        row_incl = _sub_scan_incl(row_tot)
        cum = incl + row_incl - row_tot
        counts.append(jnp.sum(m))
        ranks.append((m, cum))
    off = jnp.int32(0)
    offs = []
    for e in range(NUM_EXPERTS):
        offs.append(off)
        aligned = ((counts[e] + BM - 1) // BM) * BM
        off = off + aligned
    total_tiles = off // BM
    for e in range(NUM_EXPERTS):
        m, cum = ranks[e]
        pos = pos + jnp.where(m > 0, offs[e] + cum - 1, 0)
    pos_ref[...] = pos
    for i in range(NT):
        ei = jnp.int32(-1)
        for e in range(NUM_EXPERTS):
            ei = ei + jnp.where(i * BM >= offs[e], 1, 0)
        meta_ref[i] = jnp.maximum(ei, 0)
        meta_ref[NT + i] = jnp.where(i < total_tiles, 1, 0)
    wbc_ref[...] = jnp.broadcast_to(wcol_ref[...], (S, 16))


def _routing(ids2d, wcol):
    return pl.pallas_call(
        _routing_body,
        in_specs=[
            pl.BlockSpec(memory_space=pltpu.VMEM),
            pl.BlockSpec(memory_space=pltpu.VMEM),
        ],
        out_specs=[
            pl.BlockSpec(memory_space=pltpu.VMEM),
            pl.BlockSpec(memory_space=pltpu.SMEM),
            pl.BlockSpec(memory_space=pltpu.VMEM),
        ],
        out_shape=[
            jax.ShapeDtypeStruct((ROWS, LANES), jnp.int32),
            jax.ShapeDtypeStruct((2 * NT,), jnp.int32),
            jax.ShapeDtypeStruct((S, 16), jnp.float32),
        ],
    )(ids2d, wcol)


def _dispatch(hidden, tok, pos):
    mesh = plsc.VectorSubcoreMesh(core_axis_name="c", subcore_axis_name="s")

    @functools.partial(
        pl.kernel, mesh=mesh,
        out_type=jax.ShapeDtypeStruct((S_PAD, HIDDEN), jnp.float32),
        scratch_types=[
            pltpu.VMEM((CH,), jnp.int32),
            pltpu.VMEM((CH,), jnp.int32),
            pltpu.VMEM((CH, HIDDEN), jnp.float32),
            pltpu.SemaphoreType.DMA,
        ],
    )
    def k(hidden_hbm, tok_hbm, pos_hbm, xs_hbm, tok_v, pos_v, rows_v, sem):
        wid = lax.axis_index("s") * 2 + lax.axis_index("c")
        for j in range(SLOTS_PER_W // CH):
            base = wid * SLOTS_PER_W + j * CH
            pltpu.sync_copy(tok_hbm.at[pl.ds(base, CH)], tok_v)
            pltpu.sync_copy(pos_hbm.at[pl.ds(base, CH)], pos_v)
            pltpu.async_copy(hidden_hbm.at[tok_v], rows_v, sem).wait()
            pltpu.async_copy(rows_v, xs_hbm.at[pos_v], sem).wait()

    return k(hidden, tok, pos)


def _gemm_body(meta_ref, x_ref, wg_ref, wu_ref, w2_ref, y_ref):
    i = pl.program_id(0)

    @pl.when(meta_ref[NT + i] == 1)
    def _():
        x = x_ref[...].astype(jnp.bfloat16)
        g = lax.dot_general(x, wg_ref[0], (((1,), (1,)), ((), ())),
                            preferred_element_type=jnp.float32)
        u = lax.dot_general(x, wu_ref[0], (((1,), (1,)), ((), ())),
                            preferred_element_type=jnp.float32)
        act = (g * (1.0 / (1.0 + jnp.exp(-g))) * u).astype(jnp.bfloat16)
        y_ref[...] = lax.dot_general(act, w2_ref[0], (((1,), (1,)), ((), ())),
                                     preferred_element_type=jnp.float32)


def _gemm(meta, x_sorted, wg, wu, w2):
    grid_spec = pltpu.PrefetchScalarGridSpec(
        num_scalar_prefetch=1,
        grid=(NT,),
        in_specs=[
            pl.BlockSpec((BM, HIDDEN), lambda i, m: (i, 0)),
            pl.BlockSpec((1, INTER, HIDDEN), lambda i, m: (m[i], 0, 0)),
            pl.BlockSpec((1, INTER, HIDDEN), lambda i, m: (m[i], 0, 0)),
            pl.BlockSpec((1, HIDDEN, INTER), lambda i, m: (m[i], 0, 0)),
        ],
        out_specs=pl.BlockSpec((BM, HIDDEN), lambda i, m: (i, 0)),
    )
    return pl.pallas_call(
        _gemm_body,
        grid_spec=grid_spec,
        out_shape=jax.ShapeDtypeStruct((S_PAD, HIDDEN), jnp.float32),
        compiler_params=pltpu.CompilerParams(
            dimension_semantics=("arbitrary",)),
    )(meta, x_sorted, wg, wu, w2)


def _combine(y_sorted, pos, wbc):
    mesh = plsc.VectorSubcoreMesh(core_axis_name="c", subcore_axis_name="s")
    TOK_CH = CH // TOP_K

    @functools.partial(
        pl.kernel, mesh=mesh,
        out_type=jax.ShapeDtypeStruct((T, HIDDEN), jnp.float32),
        scratch_types=[
            pltpu.VMEM((CH,), jnp.int32),
            pltpu.VMEM((CH, 16), jnp.float32),
            pltpu.VMEM((CH, HIDDEN), jnp.float32),
            pltpu.VMEM((TOK_CH, HIDDEN), jnp.float32),
            pltpu.SemaphoreType.DMA,
        ],
    )
    def k(ys_hbm, pos_hbm, wbc_hbm, out_hbm, pos_v, w_v, ybuf, obuf, sem):
        wid = lax.axis_index("s") * 2 + lax.axis_index("c")
        for j in range(SLOTS_PER_W // CH):
            sbase = wid * SLOTS_PER_W + j * CH
            pltpu.sync_copy(pos_hbm.at[pl.ds(sbase, CH)], pos_v)
            pltpu.sync_copy(wbc_hbm.at[pl.ds(sbase, CH), :], w_v)
            pltpu.async_copy(ys_hbm.at[pos_v], ybuf, sem).wait()

            def body(t, carry):
                i0 = t * TOP_K
                i1 = i0 + 1
                w0 = w_v[i0]
                w1 = w_v[i1]
                for c in range(HIDDEN // 16):
                    sl = pl.ds(c * 16, 16)
                    obuf[t, sl] = w0 * ybuf[i0, sl] + w1 * ybuf[i1, sl]
                return carry

            lax.fori_loop(0, TOK_CH, body, 0)
            tbase = wid * (SLOTS_PER_W // TOP_K) + j * TOK_CH
            pltpu.sync_copy(obuf, out_hbm.at[pl.ds(tbase, TOK_CH)])

    return k(y_sorted, pos, wbc)


def kernel(hidden_states, topk_ids, topk_weights, w13, w2,
           num_global_tokens, max_num_tokens_per_gpu):
    ids2d = topk_ids.astype(jnp.int32).reshape(ROWS, LANES)
    wcol = topk_weights.astype(jnp.float32).reshape(S, 1)
    tok = jnp.arange(S, dtype=jnp.int32) // TOP_K

    pos2d, meta, wbc = _routing(ids2d, wcol)
    pos = pos2d.reshape(S)

    x_sorted = _dispatch(hidden_states, tok, pos)

    w13b = w13.astype(jnp.bfloat16)
    wg = w13b[:, :INTER, :]
    wu = w13b[:, INTER:, :]
    y_sorted = _gemm(meta, x_sorted, wg, wu, w2.astype(jnp.bfloat16))

    return _combine(y_sorted, pos, wbc)


# speedup vs baseline: 1.0994x; 1.0994x over previous
import functools

import jax
import jax.numpy as jnp
from jax import lax
from jax.experimental import pallas as pl
from jax.experimental.pallas import tpu as pltpu
from jax.experimental.pallas import tpu_sc as plsc

NUM_EXPERTS = 8
TOP_K = 2
HIDDEN = 1024
INTER = 1024
T = 2048
S = T * TOP_K
BM = 256
S_PAD = S + NUM_EXPERTS * BM
NT = S_PAD // BM

ROWS = 32
LANES = 128
NW = 32
SLOTS_PER_W = S // NW
CH = 64


def _lane_scan_incl(x):
    sh = 1
    col = lax.broadcasted_iota(jnp.int32, x.shape, 1)
    while sh < x.shape[1]:
        shifted = jnp.concatenate(
            [jnp.zeros((x.shape[0], sh), x.dtype), x[:, :-sh]], axis=1)
        x = x + jnp.where(col >= sh, shifted, 0)
        sh *= 2
    return x


def _sub_scan_incl(x):
    sh = 1
    row = lax.broadcasted_iota(jnp.int32, x.shape, 0)
    while sh < x.shape[0]:
        shifted = jnp.concatenate(
            [jnp.zeros((sh, x.shape[1]), x.dtype), x[:-sh, :]], axis=0)
        x = x + jnp.where(row >= sh, shifted, 0)
        sh *= 2
    return x


def _routing_body(ids_ref, wcol_ref, pos_ref, meta_ref, wbc_ref):
    ids = ids_ref[...]
    pos = jnp.zeros(ids.shape, jnp.int32)
    counts = []
    ranks = []
    for e in range(NUM_EXPERTS):
        m = (ids == e).astype(jnp.int32)
        incl = _lane_scan_incl(m)
        row_tot = jnp.broadcast_to(incl[:, LANES - 1:LANES], ids.shape)
        row_incl = _sub_scan_incl(row_tot)
        cum = incl + row_incl - row_tot
        counts.append(jnp.sum(m))
        ranks.append((m, cum))
    off = jnp.int32(0)
    offs = []
    for e in range(NUM_EXPERTS):
        offs.append(off)
        aligned = ((counts[e] + BM - 1) // BM) * BM
        off = off + aligned
    total_tiles = off // BM
    for e in range(NUM_EXPERTS):
        m, cum = ranks[e]
        pos = pos + jnp.where(m > 0, offs[e] + cum - 1, 0)
    pos_ref[...] = pos
    for i in range(NT):
        ei = jnp.int32(-1)
        for e in range(NUM_EXPERTS):
            ei = ei + jnp.where(i * BM >= offs[e], 1, 0)
        meta_ref[i] = jnp.maximum(ei, 0)
        meta_ref[NT + i] = jnp.where(i < total_tiles, 1, 0)
    wbc_ref[...] = jnp.broadcast_to(wcol_ref[...], (S, 16))


def _routing(ids2d, wcol):
    return pl.pallas_call(
        _routing_body,
        in_specs=[
            pl.BlockSpec(memory_space=pltpu.VMEM),
            pl.BlockSpec(memory_space=pltpu.VMEM),
        ],
        out_specs=[
            pl.BlockSpec(memory_space=pltpu.VMEM),
            pl.BlockSpec(memory_space=pltpu.SMEM),
            pl.BlockSpec(memory_space=pltpu.VMEM),
        ],
        out_shape=[
            jax.ShapeDtypeStruct((ROWS, LANES), jnp.int32),
            jax.ShapeDtypeStruct((2 * NT,), jnp.int32),
            jax.ShapeDtypeStruct((S, 16), jnp.float32),
        ],
    )(ids2d, wcol)


NCH = 4
CH2 = SLOTS_PER_W // NCH


def _dispatch(hidden, tok3, pos3):
    mesh = plsc.VectorSubcoreMesh(core_axis_name="c", subcore_axis_name="s")

    @functools.partial(
        pl.kernel, mesh=mesh,
        out_type=jax.ShapeDtypeStruct((S_PAD, HIDDEN), jnp.float32),
        scratch_types=[
            pltpu.VMEM((NCH, CH2), jnp.int32),
            pltpu.VMEM((NCH, CH2), jnp.int32),
            pltpu.VMEM((2, CH2, HIDDEN), jnp.float32),
            pltpu.SemaphoreType.DMA,
            pltpu.SemaphoreType.DMA,
            pltpu.SemaphoreType.DMA,
            pltpu.SemaphoreType.DMA,
        ],
    )
    def k(hidden_hbm, tok_hbm, pos_hbm, xs_hbm, tok_v, pos_v, bufs,
          gs0, gs1, ss0, ss1):
        wid = lax.axis_index("s") * 2 + lax.axis_index("c")
        gsem = [gs0, gs1]
        ssem = [ss0, ss1]
        pltpu.sync_copy(tok_hbm.at[wid], tok_v)
        pltpu.sync_copy(pos_hbm.at[wid], pos_v)
        pend_s = [None, None]
        g = pltpu.async_copy(hidden_hbm.at[tok_v.at[0]], bufs.at[0], gsem[0])
        for j in range(NCH):
            b = j % 2
            g.wait()
            if j + 1 < NCH:
                nb = (j + 1) % 2
                if pend_s[nb] is not None:
                    pend_s[nb].wait()
                g = pltpu.async_copy(
                    hidden_hbm.at[tok_v.at[j + 1]], bufs.at[nb], gsem[nb])
            pend_s[b] = pltpu.async_copy(
                bufs.at[b], xs_hbm.at[pos_v.at[j]], ssem[b])
        pend_s[0].wait()
        pend_s[1].wait()

    return k(hidden, tok3, pos3)


def _gemm_body(meta_ref, x_ref, wg_ref, wu_ref, w2_ref, y_ref):
    i = pl.program_id(0)

    @pl.when(meta_ref[NT + i] == 1)
    def _():
        x = x_ref[...]
        g = lax.dot_general(x, wg_ref[0], (((1,), (1,)), ((), ())),
                            preferred_element_type=jnp.float32)
        u = lax.dot_general(x, wu_ref[0], (((1,), (1,)), ((), ())),
                            preferred_element_type=jnp.float32)
        act = g * (1.0 / (1.0 + jnp.exp(-g))) * u
        y_ref[...] = lax.dot_general(act, w2_ref[0], (((1,), (1,)), ((), ())),
                                     preferred_element_type=jnp.float32)


def _gemm(meta, x_sorted, wg, wu, w2):
    grid_spec = pltpu.PrefetchScalarGridSpec(
        num_scalar_prefetch=1,
        grid=(NT,),
        in_specs=[
            pl.BlockSpec((BM, HIDDEN), lambda i, m: (i, 0)),
            pl.BlockSpec((1, INTER, HIDDEN), lambda i, m: (m[i], 0, 0)),
            pl.BlockSpec((1, INTER, HIDDEN), lambda i, m: (m[i], 0, 0)),
            pl.BlockSpec((1, HIDDEN, INTER), lambda i, m: (m[i], 0, 0)),
        ],
        out_specs=pl.BlockSpec((BM, HIDDEN), lambda i, m: (i, 0)),
    )
    return pl.pallas_call(
        _gemm_body,
        grid_spec=grid_spec,
        out_shape=jax.ShapeDtypeStruct((S_PAD, HIDDEN), jnp.float32),
        compiler_params=pltpu.CompilerParams(
            dimension_semantics=("arbitrary",)),
    )(meta, x_sorted, wg, wu, w2)


def _combine(y_sorted, pos3, wbc4):
    mesh = plsc.VectorSubcoreMesh(core_axis_name="c", subcore_axis_name="s")
    TOK_CH = CH2 // TOP_K

    @functools.partial(
        pl.kernel, mesh=mesh,
        out_type=jax.ShapeDtypeStruct((T, HIDDEN), jnp.float32),
        scratch_types=[
            pltpu.VMEM((NCH, CH2), jnp.int32),
            pltpu.VMEM((NCH, CH2, 16), jnp.float32),
            pltpu.VMEM((2, CH2, HIDDEN), jnp.float32),
            pltpu.VMEM((2, TOK_CH, HIDDEN), jnp.float32),
            pltpu.SemaphoreType.DMA,
            pltpu.SemaphoreType.DMA,
            pltpu.SemaphoreType.DMA,
            pltpu.SemaphoreType.DMA,
        ],
    )
    def k(ys_hbm, pos_hbm, wbc_hbm, out_hbm, pos_v, w_v, ybufs, obufs,
          gs0, gs1, os0, os1):
        wid = lax.axis_index("s") * 2 + lax.axis_index("c")
        gsem = [gs0, gs1]
        osem = [os0, os1]
        pltpu.sync_copy(pos_hbm.at[wid], pos_v)
        pltpu.sync_copy(wbc_hbm.at[wid], w_v)
        pend_o = [None, None]
        g = pltpu.async_copy(ys_hbm.at[pos_v.at[0]], ybufs.at[0], gsem[0])
        for j in range(NCH):
            b = j % 2
            g.wait()
            if j + 1 < NCH:
                nb = (j + 1) % 2
                g = pltpu.async_copy(
                    ys_hbm.at[pos_v.at[j + 1]], ybufs.at[nb], gsem[nb])
            if pend_o[b] is not None:
                pend_o[b].wait()

            def body(t, carry):
                i0 = t * TOP_K
                i1 = i0 + 1
                w0 = w_v[j, i0]
                w1 = w_v[j, i1]
                for c in range(HIDDEN // 16):
                    sl = pl.ds(c * 16, 16)
                    obufs[b, t, sl] = (w0 * ybufs[b, i0, sl]
                                       + w1 * ybufs[b, i1, sl])
                return carry

            lax.fori_loop(0, TOK_CH, body, 0)
            tbase = wid * (SLOTS_PER_W // TOP_K) + j * TOK_CH
            pend_o[b] = pltpu.async_copy(
                obufs.at[b], out_hbm.at[pl.ds(tbase, TOK_CH)], osem[b])
        pend_o[0].wait()
        pend_o[1].wait()

    return k(y_sorted, pos3, wbc4)


def kernel(hidden_states, topk_ids, topk_weights, w13, w2,
           num_global_tokens, max_num_tokens_per_gpu):
    ids2d = topk_ids.astype(jnp.int32).reshape(ROWS, LANES)
    wcol = topk_weights.astype(jnp.float32).reshape(S, 1)
    tok = jnp.arange(S, dtype=jnp.int32) // TOP_K

    pos2d, meta, wbc = _routing(ids2d, wcol)
    pos3 = pos2d.reshape(NW, NCH, CH2)
    tok3 = tok.reshape(NW, NCH, CH2)
    wbc4 = wbc.reshape(NW, NCH, CH2, 16)

    x_sorted = _dispatch(hidden_states, tok3, pos3)

    wg = w13[:, :INTER, :]
    wu = w13[:, INTER:, :]
    y_sorted = _gemm(meta, x_sorted, wg, wu, w2)

    return _combine(y_sorted, pos3, wbc4)
